# baseline (speedup 1.0000x reference)
import functools
import math

import jax
import jax.numpy as jnp
from jax import lax
from jax.experimental import pallas as pl
from jax.experimental.pallas import tpu as pltpu
from jax.experimental.pallas import tpu_sc as plsc

_NC, _NS = 2, 16
_NW = _NC * _NS
_B, _T, _V, _D = 16384, 200, 100000, 32
_BT = _B // 128
_BT_PER_W = _BT // _NW
_NBUF = 5
_TPAD = 129


def _scale_body(t_ref, o_ref, *, scale):
    o_ref[...] = t_ref[...] * scale


def _scale_table(table, scale):
    v, d = table.shape
    wide = table.reshape(v * d // 512, 512)
    scaled = pl.pallas_call(
        functools.partial(_scale_body, scale=scale),
        out_shape=jax.ShapeDtypeStruct(wide.shape, table.dtype),
    )(wide)
    return scaled.reshape(v, d)


def _make_gather_kernel():
    mesh = plsc.VectorSubcoreMesh(
        core_axis_name="c", subcore_axis_name="s",
        num_cores=_NC, num_subcores=_NS,
    )

    @functools.partial(
        pl.kernel,
        out_type=jax.ShapeDtypeStruct((_T, _D // 8, _BT, 8, 128), jnp.float32),
        mesh=mesh,
        scratch_types=[
            pltpu.VMEM((_T, 128), jnp.int32),
            pltpu.VMEM((_NBUF, 128, _D), jnp.float32),
            pltpu.VMEM((_NBUF, _D, _TPAD), jnp.float32),
            pltpu.SemaphoreType.DMA((_NBUF,)),
            pltpu.SemaphoreType.DMA((_NBUF,)),
        ],
        compiler_params=pltpu.CompilerParams(
            use_tc_tiling_on_sc=False, needs_layout_passes=False),
    )
    def gather_kernel(table_hbm, tokt_hbm, out_hbm, panel, rows, tr,
                      sem_g, sem_st):
        wid = lax.axis_index("s") * _NC + lax.axis_index("c")
        iotas = [lax.iota(jnp.int32, 16) + 16 * v for v in range(8)]

        def gather_copy(t, s):
            return pltpu.make_async_copy(
                table_hbm.at[panel.at[t]], rows.at[s], sem_g.at[s])

        def store_chunk(t, s, dt, bt):
            return pltpu.make_async_copy(
                tr.at[s].at[pl.ds(dt * 8, 8), pl.ds(0, 128)],
                out_hbm.at[t].at[dt].at[wid * _BT_PER_W + bt],
                sem_st.at[s])

        def transpose(s):
            for b in range(128):
                col = jnp.full((16,), b, jnp.int32)
                for h in range(2):
                    vals = rows[s, b, pl.ds(16 * h, 16)]
                    plsc.store_scatter(tr.at[s], [iotas[h], col], vals)

        for bt in range(_BT_PER_W):
            b0 = (wid * _BT_PER_W + bt) * 128
            pltpu.sync_copy(
                tokt_hbm.at[pl.ds(0, _T), pl.ds(b0, 128)], panel)
            for s in range(_NBUF - 1):
                gather_copy(s, s).start()

            @pl.loop(0, _T)
            def _(u):
                s = lax.rem(u, _NBUF)

                @pl.when(u + _NBUF - 1 < _T)
                def _():
                    gather_copy(u + _NBUF - 1,
                                lax.rem(u + _NBUF - 1, _NBUF)).start()

                gather_copy(u, s).wait()

                @pl.when(u >= _NBUF)
                def _():
                    for dt in range(_D // 8):
                        store_chunk(u - _NBUF, s, dt, bt).wait()

                transpose(s)

                for dt in range(_D // 8):
                    store_chunk(u, s, dt, bt).start()

            for s in range(_NBUF):
                for dt in range(_D // 8):
                    store_chunk(_T - _NBUF + s, s, dt, bt).wait()

    return gather_kernel


def kernel(tokens, table):
    scaled = _scale_table(table, math.sqrt(_D))
    tokt = jnp.swapaxes(tokens, 0, 1)
    out5 = _make_gather_kernel()(scaled, tokt)
    return out5.transpose(2, 4, 0, 1, 3).reshape(_B, _T, _D)


# speedup vs baseline: 2.4425x; 2.4425x over previous
import functools
import math

import jax
import jax.numpy as jnp
from jax import lax
from jax.experimental import pallas as pl
from jax.experimental.pallas import tpu as pltpu
from jax.experimental.pallas import tpu_sc as plsc

_NC, _NS = 2, 16
_NW = _NC * _NS
_B, _T, _V, _D = 16384, 200, 100000, 32
_BT = _B // 128
_BT_PER_W = _BT // _NW
_NBUF = 5
_TPAD = 129


def _scale_body(t_ref, o_ref, *, scale):
    o_ref[...] = t_ref[...] * scale


def _scale_table(table, scale):
    v, d = table.shape
    wide = table.reshape(v * d // 512, 512)
    scaled = pl.pallas_call(
        functools.partial(_scale_body, scale=scale),
        out_shape=jax.ShapeDtypeStruct(wide.shape, table.dtype),
    )(wide)
    return scaled.reshape(v, d)


def _make_gather_kernel():
    mesh = plsc.VectorSubcoreMesh(
        core_axis_name="c", subcore_axis_name="s",
        num_cores=_NC, num_subcores=_NS,
    )

    @functools.partial(
        pl.kernel,
        out_type=jax.ShapeDtypeStruct((_T, _D // 8, _BT, 8, 128), jnp.float32),
        mesh=mesh,
        scratch_types=[
            pltpu.VMEM((_T, 128), jnp.int32),
            pltpu.VMEM((_NBUF, 128, _D), jnp.float32),
            pltpu.VMEM((_NBUF, _D, _TPAD), jnp.float32),
            pltpu.SemaphoreType.DMA((_NBUF,)),
            pltpu.SemaphoreType.DMA((_NBUF,)),
        ],
        compiler_params=pltpu.CompilerParams(
            use_tc_tiling_on_sc=False, needs_layout_passes=False),
    )
    def gather_kernel(table_hbm, tokt_hbm, out_hbm, panel, rows, tr,
                      sem_g, sem_st):
        wid = lax.axis_index("s") * _NC + lax.axis_index("c")
        iotas = [lax.iota(jnp.int32, 16) + 16 * v for v in range(8)]

        def gather_copy(t, s):
            return pltpu.make_async_copy(
                table_hbm.at[panel.at[t]], rows.at[s], sem_g.at[s])

        def store_chunk(t, s, dt, bt):
            return pltpu.make_async_copy(
                tr.at[s].at[pl.ds(dt * 8, 8), pl.ds(0, 128)],
                out_hbm.at[t].at[dt].at[wid * _BT_PER_W + bt],
                sem_st.at[s])

        def transpose(s):
            @plsc.parallel_loop(0, 128, unroll=32)
            def _(b):
                col = jnp.full((16,), 0, jnp.int32) + b
                for h in range(2):
                    vals = rows[s, b, pl.ds(16 * h, 16)]
                    plsc.store_scatter(tr.at[s], [iotas[h], col], vals)

        for bt in range(_BT_PER_W):
            b0 = (wid * _BT_PER_W + bt) * 128
            pltpu.sync_copy(
                tokt_hbm.at[pl.ds(0, _T), pl.ds(b0, 128)], panel)
            for s in range(_NBUF - 1):
                gather_copy(s, s).start()

            @pl.loop(0, _T)
            def _(u):
                s = lax.rem(u, _NBUF)

                @pl.when(u + _NBUF - 1 < _T)
                def _():
                    gather_copy(u + _NBUF - 1,
                                lax.rem(u + _NBUF - 1, _NBUF)).start()

                gather_copy(u, s).wait()

                @pl.when(u >= _NBUF)
                def _():
                    for dt in range(_D // 8):
                        store_chunk(u - _NBUF, s, dt, bt).wait()

                transpose(s)

                for dt in range(_D // 8):
                    store_chunk(u, s, dt, bt).start()

            for s in range(_NBUF):
                for dt in range(_D // 8):
                    store_chunk(_T - _NBUF + s, s, dt, bt).wait()

    return gather_kernel


def kernel(tokens, table):
    scaled = _scale_table(table, math.sqrt(_D))
    tokt = jnp.swapaxes(tokens, 0, 1)
    out5 = _make_gather_kernel()(scaled, tokt)
    return out5.transpose(2, 4, 0, 1, 3).reshape(_B, _T, _D)


# speedup vs baseline: 2.5391x; 1.0396x over previous
import functools
import math

import jax
import jax.numpy as jnp
from jax import lax
from jax.experimental import pallas as pl
from jax.experimental.pallas import tpu as pltpu
from jax.experimental.pallas import tpu_sc as plsc

_NC, _NS = 2, 16
_NW = _NC * _NS
_B, _T, _V, _D = 16384, 200, 100000, 32
_BT = _B // 128
_BT_PER_W = _BT // _NW
_NBUF = 8
_TPAD = 129


def _scale_body(t_ref, o_ref, *, scale):
    o_ref[...] = t_ref[...] * scale


def _scale_table(table, scale):
    v, d = table.shape
    wide = table.reshape(v * d // 512, 512)
    scaled = pl.pallas_call(
        functools.partial(_scale_body, scale=scale),
        out_shape=jax.ShapeDtypeStruct(wide.shape, table.dtype),
    )(wide)
    return scaled.reshape(v, d)


def _make_gather_kernel():
    mesh = plsc.VectorSubcoreMesh(
        core_axis_name="c", subcore_axis_name="s",
        num_cores=_NC, num_subcores=_NS,
    )

    @functools.partial(
        pl.kernel,
        out_type=jax.ShapeDtypeStruct((_T, _D // 8, _BT, 8, 128), jnp.float32),
        mesh=mesh,
        scratch_types=[
            pltpu.VMEM((_T, 128), jnp.int32),
            pltpu.VMEM((_NBUF, 128, _D), jnp.float32),
            pltpu.VMEM((_NBUF, _D, _TPAD), jnp.float32),
            pltpu.SemaphoreType.DMA((_NBUF,)),
            pltpu.SemaphoreType.DMA((_NBUF,)),
        ],
        compiler_params=pltpu.CompilerParams(
            use_tc_tiling_on_sc=False, needs_layout_passes=False),
    )
    def gather_kernel(table_hbm, tokt_hbm, out_hbm, panel, rows, tr,
                      sem_g, sem_st):
        wid = lax.axis_index("s") * _NC + lax.axis_index("c")
        iotas = [lax.iota(jnp.int32, 16) + 16 * v for v in range(8)]

        def gather_copy(t, s):
            return pltpu.make_async_copy(
                table_hbm.at[panel.at[t]], rows.at[s], sem_g.at[s])

        def store_chunk(t, s, dt, bt):
            return pltpu.make_async_copy(
                tr.at[s].at[pl.ds(dt * 8, 8), pl.ds(0, 128)],
                out_hbm.at[t].at[dt].at[wid * _BT_PER_W + bt],
                sem_st.at[s])

        def transpose(s):
            @plsc.parallel_loop(0, 128, unroll=32)
            def _(b):
                col = jnp.full((16,), 0, jnp.int32) + b
                for h in range(2):
                    vals = rows[s, b, pl.ds(16 * h, 16)]
                    plsc.store_scatter(tr.at[s], [iotas[h], col], vals)

        for bt in range(_BT_PER_W):
            b0 = (wid * _BT_PER_W + bt) * 128
            pltpu.sync_copy(
                tokt_hbm.at[pl.ds(0, _T), pl.ds(b0, 128)], panel)
            for s in range(_NBUF - 1):
                gather_copy(s, s).start()

            @pl.loop(0, _T)
            def _(u):
                s = lax.rem(u, _NBUF)

                @pl.when(u + _NBUF - 1 < _T)
                def _():
                    gather_copy(u + _NBUF - 1,
                                lax.rem(u + _NBUF - 1, _NBUF)).start()

                gather_copy(u, s).wait()

                @pl.when(u >= _NBUF)
                def _():
                    for dt in range(_D // 8):
                        store_chunk(u - _NBUF, s, dt, bt).wait()

                transpose(s)

                for dt in range(_D // 8):
                    store_chunk(u, s, dt, bt).start()

            for s in range(_NBUF):
                for dt in range(_D // 8):
                    store_chunk(_T - _NBUF + s, s, dt, bt).wait()

    return gather_kernel


def kernel(tokens, table):
    scaled = _scale_table(table, math.sqrt(_D))
    tokt = jnp.swapaxes(tokens, 0, 1)
    out5 = _make_gather_kernel()(scaled, tokt)
    return out5.transpose(2, 4, 0, 1, 3).reshape(_B, _T, _D)


# speedup vs baseline: 2.5643x; 1.0099x over previous
import functools
import math

import jax
import jax.numpy as jnp
from jax import lax
from jax.experimental import pallas as pl
from jax.experimental.pallas import tpu as pltpu
from jax.experimental.pallas import tpu_sc as plsc

_NC, _NS = 2, 16
_NW = _NC * _NS
_B, _T, _V, _D = 16384, 200, 100000, 32
_BT = _B // 128
_BT_PER_W = _BT // _NW
_NBUF = 8
_TPAD = 129


def _scale_body(t_ref, o_ref, *, scale):
    o_ref[...] = t_ref[...] * scale


def _scale_table(table, scale):
    v, d = table.shape
    wide = table.reshape(v * d // 512, 512)
    scaled = pl.pallas_call(
        functools.partial(_scale_body, scale=scale),
        out_shape=jax.ShapeDtypeStruct(wide.shape, table.dtype),
    )(wide)
    return scaled.reshape(v, d)


def _make_gather_kernel():
    mesh = plsc.VectorSubcoreMesh(
        core_axis_name="c", subcore_axis_name="s",
        num_cores=_NC, num_subcores=_NS,
    )

    @functools.partial(
        pl.kernel,
        out_type=jax.ShapeDtypeStruct((_T, _D // 8, _BT, 8, 128), jnp.float32),
        mesh=mesh,
        scratch_types=[
            pltpu.VMEM((2, _T, 128), jnp.int32),
            pltpu.VMEM((_NBUF, 128, _D), jnp.float32),
            pltpu.VMEM((_NBUF, _D, _TPAD), jnp.float32),
            pltpu.SemaphoreType.DMA((_NBUF,)),
            pltpu.SemaphoreType.DMA((_NBUF,)),
            pltpu.SemaphoreType.DMA,
        ],
        compiler_params=pltpu.CompilerParams(
            use_tc_tiling_on_sc=False, needs_layout_passes=False),
    )
    def gather_kernel(table_hbm, tokt_hbm, out_hbm, panel, rows, tr,
                      sem_g, sem_st, sem_p):
        wid = lax.axis_index("s") * _NC + lax.axis_index("c")
        iotas = [lax.iota(jnp.int32, 16) + 16 * v for v in range(8)]

        def panel_copy(bt):
            b0 = (wid * _BT_PER_W + bt) * 128
            return pltpu.make_async_copy(
                tokt_hbm.at[pl.ds(0, _T), pl.ds(b0, 128)],
                panel.at[bt % 2], sem_p)

        def gather_copy(t, s, pb):
            return pltpu.make_async_copy(
                table_hbm.at[panel.at[pb].at[t]], rows.at[s], sem_g.at[s])

        def store_chunk(t, s, dt, bt):
            return pltpu.make_async_copy(
                tr.at[s].at[pl.ds(dt * 8, 8), pl.ds(0, 128)],
                out_hbm.at[t].at[dt].at[wid * _BT_PER_W + bt],
                sem_st.at[s])

        def transpose(s):
            @plsc.parallel_loop(0, 128, unroll=32)
            def _(b):
                col = jnp.full((16,), 0, jnp.int32) + b
                for h in range(2):
                    vals = rows[s, b, pl.ds(16 * h, 16)]
                    plsc.store_scatter(tr.at[s], [iotas[h], col], vals)

        panel_copy(0).start()
        for bt in range(_BT_PER_W):
            pb = bt % 2
            panel_copy(bt).wait()
            if bt + 1 < _BT_PER_W:
                panel_copy(bt + 1).start()
            for s in range(_NBUF - 1):
                gather_copy(s, s, pb).start()

            @pl.loop(0, _T)
            def _(u):
                s = lax.rem(u, _NBUF)

                @pl.when(u + _NBUF - 1 < _T)
                def _():
                    gather_copy(u + _NBUF - 1,
                                lax.rem(u + _NBUF - 1, _NBUF), pb).start()

                gather_copy(u, s, pb).wait()

                @pl.when(u >= _NBUF)
                def _():
                    for dt in range(_D // 8):
                        store_chunk(u - _NBUF, s, dt, bt).wait()

                transpose(s)

                for dt in range(_D // 8):
                    store_chunk(u, s, dt, bt).start()

            for s in range(_NBUF):
                for dt in range(_D // 8):
                    store_chunk(_T - _NBUF + s, s, dt, bt).wait()

    return gather_kernel


def kernel(tokens, table):
    scaled = _scale_table(table, math.sqrt(_D))
    tokt = jnp.swapaxes(tokens, 0, 1)
    out5 = _make_gather_kernel()(scaled, tokt)
    return out5.transpose(2, 4, 0, 1, 3).reshape(_B, _T, _D)
